# baseline (speedup 1.0000x reference)
import functools

import jax
import jax.numpy as jnp
from jax.experimental import pallas as pl
from jax.experimental.pallas import tpu as pltpu

_ROW_TILE = 1024
_NBUF = 2
_VMEM_LIMIT_BYTES = 48 << 20


def _stream_pool_kernel(x_hbm, o_ref, buf, sems, *, nb, ch, tr, inv_count):
    core = pl.program_id(0)
    base = core * nb
    n_chunks = nb * ch

    def start(k):
        lb, c = divmod(k, ch)
        slot = k % _NBUF
        pltpu.make_async_copy(
            x_hbm.at[base + lb, pl.ds(c * tr, tr), :],
            buf.at[slot],
            sems.at[slot],
        ).start()

    def wait(k):
        slot = k % _NBUF
        pltpu.make_async_copy(buf.at[slot], buf.at[slot], sems.at[slot]).wait()

    for k in range(min(_NBUF, n_chunks)):
        start(k)

    for lb in range(nb):
        acc = None
        for c in range(ch):
            k = lb * ch + c
            wait(k)
            x = buf[k % _NBUF]
            part = jnp.sum(x.reshape(-1, 8, x.shape[-1]), axis=0)
            acc = part if acc is None else acc + part
            if k + _NBUF < n_chunks:
                start(k + _NBUF)
        total = jnp.sum(acc, axis=0, keepdims=True) * inv_count
        o_ref[lb : lb + 1, :] = total.astype(o_ref.dtype)


def kernel(tokens, outputs):
    del tokens
    B, S1, S2, D = outputs.shape
    R = S1 * S2
    x = outputs.reshape(B, R, D)

    ncores = 2 if B % 2 == 0 else 1
    nb = B // ncores
    tr = _ROW_TILE if (R % _ROW_TILE == 0 and R >= _ROW_TILE) else R
    ch = R // tr

    out = pl.pallas_call(
        functools.partial(
            _stream_pool_kernel, nb=nb, ch=ch, tr=tr, inv_count=1.0 / R
        ),
        out_shape=jax.ShapeDtypeStruct((ncores, nb, D), outputs.dtype),
        grid_spec=pltpu.PrefetchScalarGridSpec(
            num_scalar_prefetch=0,
            grid=(ncores,),
            in_specs=[pl.BlockSpec(memory_space=pl.ANY)],
            out_specs=pl.BlockSpec(
                (pl.Squeezed(), nb, D), lambda c: (c, 0, 0)
            ),
            scratch_shapes=[
                pltpu.VMEM((_NBUF, tr, D), outputs.dtype),
                pltpu.SemaphoreType.DMA((_NBUF,)),
            ],
        ),
        compiler_params=pltpu.CompilerParams(
            dimension_semantics=("parallel",),
            vmem_limit_bytes=_VMEM_LIMIT_BYTES,
        ),
    )(x)
    return out.reshape(B, D)


# speedup vs baseline: 1.0235x; 1.0235x over previous
import functools

import jax
import jax.numpy as jnp
from jax.experimental import pallas as pl
from jax.experimental.pallas import tpu as pltpu

_ROW_TILE = 1024
_VMEM_LIMIT_BYTES = 48 << 20


def _pool_kernel(x_ref, o_ref, acc_ref, *, inv_count):
    j = pl.program_id(1)
    x = x_ref[...]
    tile_part = jnp.sum(x.reshape(-1, 8, x.shape[-1]), axis=0)

    @pl.when(j == 0)
    def _():
        acc_ref[...] = tile_part

    @pl.when(j != 0)
    def _():
        acc_ref[...] += tile_part

    @pl.when(j == pl.num_programs(1) - 1)
    def _():
        total = jnp.sum(acc_ref[...], axis=0, keepdims=True)
        o_ref[0] = (total * inv_count).astype(o_ref.dtype)


def kernel(tokens, outputs):
    del tokens
    B, S1, S2, D = outputs.shape
    R = S1 * S2
    x = outputs.reshape(B, R, D)

    tr = _ROW_TILE
    if R % tr != 0 or tr % 8 != 0:
        tr = R

    out = pl.pallas_call(
        functools.partial(_pool_kernel, inv_count=1.0 / R),
        out_shape=jax.ShapeDtypeStruct((B, 1, D), outputs.dtype),
        grid_spec=pltpu.PrefetchScalarGridSpec(
            num_scalar_prefetch=0,
            grid=(B, R // tr),
            in_specs=[
                pl.BlockSpec((pl.Squeezed(), tr, D), lambda b, j: (b, j, 0))
            ],
            out_specs=pl.BlockSpec((1, 1, D), lambda b, j: (b, 0, 0)),
            scratch_shapes=[pltpu.VMEM((8, D), jnp.float32)],
        ),
        compiler_params=pltpu.CompilerParams(
            dimension_semantics=("parallel", "arbitrary"),
            vmem_limit_bytes=_VMEM_LIMIT_BYTES,
        ),
    )(x)
    return out[:, 0, :]
